# baseline (speedup 1.0000x reference)
import math

import numpy as np
import jax
import jax.numpy as jnp
from jax import lax
from jax.experimental import pallas as pl
from jax.experimental.pallas import tpu as pltpu
from jax.experimental.pallas import tpu_sc as plsc

N = 10000
E = 160000
B = 64
CTX = 512
PED = 240
HID = 128
OUT = 1024

NC = 2
NS = 16
NW = NC * NS
CH = 128
E_PAD = 163840
ESL = E_PAD // 2
N_PAD = 10112
RPS = N_PAD // NS
DUMMY = N + 8

PEP = PED // 2
PEB_W = 128

NB = 2000
EB = 2048


def _sc_mesh():
    return plsc.VectorSubcoreMesh(core_axis_name="c", subcore_axis_name="s",
                                  num_cores=NC, num_subcores=NS)


def _make_gather(n_edges):
    epw = n_edges // NW
    nchunk = epw // CH
    assert nchunk % 2 == 0

    def body(table_hbm, idx_hbm, out_hbm, tab_sh, i0, i1, r0, r1, *sems):
        idxb = (i0, i1)
        rows = (r0, r1)
        isem = sems[0:2]
        gsem = sems[2:4]
        wsem = sems[4:6]
        c = lax.axis_index("c")
        s = lax.axis_index("s")
        base = (s * NC + c) * epw

        pltpu.sync_copy(table_hbm.at[pl.ds(s * RPS, RPS)],
                        tab_sh.at[pl.ds(s * RPS, RPS)])

        def issue_idx(ci, b):
            pltpu.async_copy(idx_hbm.at[pl.ds(base + ci * CH, CH)], idxb[b],
                             isem[b])

        def wait_idx(ci, b):
            pltpu.make_async_copy(idx_hbm.at[pl.ds(base + ci * CH, CH)], idxb[b],
                                  isem[b]).wait()

        def issue_gather(ci, b):
            pltpu.async_copy(tab_sh.at[idxb[b]], rows[b], gsem[b])

        def wait_gather(ci, b):
            pltpu.make_async_copy(tab_sh.at[idxb[b]], rows[b], gsem[b]).wait()

        def issue_wb(ci, b):
            pltpu.async_copy(rows[b], out_hbm.at[pl.ds(base + ci * CH, CH)],
                             wsem[b])

        def wait_wb(ci, b):
            pltpu.make_async_copy(rows[b], out_hbm.at[pl.ds(base + ci * CH, CH)],
                                  wsem[b]).wait()

        issue_idx(0, 0)
        issue_idx(1, 1)
        plsc.subcore_barrier()

        def loop(g):
            for b in range(2):
                ci = g + b

                @pl.when(ci >= 2)
                def _():
                    wait_wb(ci - 2, b)

                wait_idx(ci, b)
                issue_gather(ci, b)
                wait_gather(ci, b)
                issue_wb(ci, b)

                ci2 = ci + 2

                @pl.when(ci2 < nchunk)
                def _():
                    issue_idx(ci2, b)

        pl.loop(0, nchunk, step=2)(loop)
        wait_wb(nchunk - 2, 0)
        wait_wb(nchunk - 1, 1)

    kern = pl.kernel(
        body,
        out_type=jax.ShapeDtypeStruct((n_edges, PEB_W), jnp.float32),
        mesh=_sc_mesh(),
        scratch_types=(
            [pltpu.VMEM_SHARED((N_PAD, PEB_W), jnp.float32)]
            + [pltpu.VMEM((CH,), jnp.int32)] * 2
            + [pltpu.VMEM((CH, PEB_W), jnp.float32)] * 2
            + [pltpu.SemaphoreType.DMA] * 6
        ),
    )
    return kern


def _make_scatter(slab_off):
    epw = ESL // NW
    nchunk = epw // CH

    def body(m_hbm, dst_hbm, zeros_hbm, out_hbm, *refs):
        idxb = refs[0:3]
        rows = refs[3:6]
        acc_sh = refs[6]
        sems = refs[7:]
        isem = sems[0:3]
        lsem = sems[3:6]
        ssem = sems[6:9]
        c = lax.axis_index("c")
        s = lax.axis_index("s")
        pltpu.sync_copy(zeros_hbm.at[pl.ds(s * RPS, RPS)],
                        acc_sh.at[pl.ds(s * RPS, RPS)])
        base = c * (ESL // NC) + s * epw

        def issue_idx(ci, b):
            pltpu.async_copy(dst_hbm.at[pl.ds(slab_off + base + ci * CH, CH)],
                             idxb[b], isem[b])

        def wait_idx(ci, b):
            pltpu.make_async_copy(dst_hbm.at[pl.ds(slab_off + base + ci * CH, CH)],
                                  idxb[b], isem[b]).wait()

        def issue_load(ci, b):
            pltpu.async_copy(m_hbm.at[pl.ds(base + ci * CH, CH)], rows[b], lsem[b])

        def wait_load(ci, b):
            pltpu.make_async_copy(m_hbm.at[pl.ds(base + ci * CH, CH)], rows[b],
                                  lsem[b]).wait()

        def issue_scat(ci, b):
            pltpu.async_copy(rows[b], acc_sh.at[idxb[b]], ssem[b], add=True)

        def wait_scat(ci, b):
            pltpu.make_async_copy(rows[b], acc_sh.at[idxb[b]], ssem[b]).wait()

        issue_idx(0, 0)
        issue_load(0, 0)
        issue_idx(1, 1)
        issue_load(1, 1)
        plsc.subcore_barrier()

        def loop(g):
            for b in range(3):
                ci = g + b
                wait_idx(ci, b)
                wait_load(ci, b)
                issue_scat(ci, b)
                cl = ci + 2
                b2 = (b + 2) % 3

                @pl.when(cl < nchunk)
                def _():
                    @pl.when(cl >= 3)
                    def _():
                        wait_scat(cl - 3, b2)

                    issue_idx(cl, b2)
                    issue_load(cl, b2)

        pl.loop(0, nchunk - 2, step=3)(loop)
        wait_idx(nchunk - 2, 0)
        wait_load(nchunk - 2, 0)
        issue_scat(nchunk - 2, 0)
        wait_idx(nchunk - 1, 1)
        wait_load(nchunk - 1, 1)
        issue_scat(nchunk - 1, 1)
        wait_scat(nchunk - 3, 2)
        wait_scat(nchunk - 2, 0)
        wait_scat(nchunk - 1, 1)
        plsc.subcore_barrier()
        pltpu.sync_copy(acc_sh.at[pl.ds(s * RPS, RPS)],
                        out_hbm.at[c, pl.ds(s * RPS, RPS)])

    kern = pl.kernel(
        body,
        out_type=jax.ShapeDtypeStruct((NC, N_PAD, HID), jnp.float32),
        mesh=_sc_mesh(),
        scratch_types=(
            [pltpu.VMEM((CH,), jnp.int32)] * 3
            + [pltpu.VMEM((CH, HID), jnp.float32)] * 3
            + [pltpu.VMEM_SHARED((N_PAD, HID), jnp.float32)]
            + [pltpu.SemaphoreType.DMA] * 9
        ),
    )
    return kern


_HI_MASK = np.uint32(0xFFFF0000)
_RND = np.uint32(0x8000)


def _unpack_pe(words_f32):
    w = lax.bitcast_convert_type(words_f32, jnp.uint32)
    lo = lax.bitcast_convert_type(w & _HI_MASK, jnp.float32)
    hi = lax.bitcast_convert_type(lax.shift_left(w, np.uint32(16)), jnp.float32)
    return lo, hi


def _prep_body(x_ref, pos_ref, batch_ref, w0_ref, b0_ref, h0_ref, peb_ref):
    xi = x_ref[...]
    oh = (xi == lax.broadcasted_iota(jnp.int32, (NB, 118), 1)).astype(jnp.float32)
    h0_ref[...] = jnp.dot(oh, w0_ref[...], preferred_element_type=jnp.float32) + b0_ref[...]

    dt = jnp.exp(lax.broadcasted_iota(jnp.int32, (1, PED // 6), 1).astype(jnp.float32)
                 * (-math.log(10000.0) / (PED // 2)))
    pos = pos_ref[...]
    parts = []
    for i in range(3):
        sarg = pos[:, i:i + 1] * dt
        parts.append(jnp.sin(sarg))
        parts.append(jnp.cos(sarg))
    lo = jnp.concatenate(parts[:3], axis=1)
    hi = jnp.concatenate(parts[3:], axis=1)
    ulo = lax.bitcast_convert_type(lo, jnp.uint32)
    uhi = lax.bitcast_convert_type(hi, jnp.uint32)
    word = ((ulo + _RND) & _HI_MASK) | lax.shift_right_logical(uhi + _RND,
                                                               np.uint32(16))
    packed = lax.bitcast_convert_type(word, jnp.float32)
    bf = batch_ref[...].astype(jnp.float32)
    pad = jnp.zeros((NB, PEB_W - PEP - 1), jnp.float32)
    peb_ref[...] = jnp.concatenate([packed, bf, pad], axis=1)


def _tc_prep(x, pos_p, batch2, w0, b02):
    grid = N // NB
    return pl.pallas_call(
        _prep_body,
        grid=(grid,),
        in_specs=[
            pl.BlockSpec((NB, 1), lambda i: (i, 0)),
            pl.BlockSpec((NB, 8), lambda i: (i, 0)),
            pl.BlockSpec((NB, 1), lambda i: (i, 0)),
            pl.BlockSpec((118, HID), lambda i: (0, 0)),
            pl.BlockSpec((1, HID), lambda i: (0, 0)),
        ],
        out_specs=[
            pl.BlockSpec((NB, HID), lambda i: (i, 0)),
            pl.BlockSpec((NB, PEB_W), lambda i: (i, 0)),
        ],
        out_shape=[
            jax.ShapeDtypeStruct((N_PAD, HID), jnp.float32),
            jax.ShapeDtypeStruct((N_PAD, PEB_W), jnp.float32),
        ],
    )(x, pos_p, batch2, w0, b02)


def _bdot(a, b):
    return jnp.dot(a.astype(jnp.bfloat16), b.astype(jnp.bfloat16),
                   preferred_element_type=jnp.float32)


MW = 384


def _msg_body(gh_ref, pes_ref, ea_ref, ctx_ref,
              wem_ref, wec_ref, bem_ref, bec_ref,
              wnm_ref, wnc_ref, m_ref):
    ea = ea_ref[...]
    gh = gh_ref[...]
    pes = pes_ref[...]
    pe_lo, pe_hi = _unpack_pe(pes[:, :PEP])
    gbf = pes[:, PEP:PEP + 1]

    z8 = jnp.zeros((EB, 8), jnp.float32)
    base = jnp.concatenate([gh, pe_lo, z8, pe_hi, z8], axis=1)
    pre_m = jax.nn.relu(base + jnp.dot(ea, wem_ref[...],
                                       preferred_element_type=jnp.float32)
                        + bem_ref[...])
    oh = (gbf.astype(jnp.int32) == lax.broadcasted_iota(jnp.int32, (EB, B), 1)
          ).astype(jnp.float32)
    cg = _bdot(oh, ctx_ref[...])
    pre_c = jax.nn.relu(cg + jnp.dot(ea, wec_ref[...], preferred_element_type=jnp.float32)
                        + bec_ref[...])

    m_ref[...] = _bdot(pre_m, wnm_ref[...]) + _bdot(pre_c, wnc_ref[...])


def _tc_msg(slab, gh, pes, ea_p, ctx, weights):
    grid = ESL // EB
    blk0 = slab * (ESL // EB)

    def whole(shape):
        return pl.BlockSpec(shape, lambda i: tuple(0 for _ in shape))

    return pl.pallas_call(
        _msg_body,
        grid=(grid,),
        in_specs=[
            pl.BlockSpec((EB, HID), lambda i: (i, 0)),
            pl.BlockSpec((EB, PEB_W), lambda i: (i + blk0, 0)),
            pl.BlockSpec((EB, 5), lambda i: (i + blk0, 0)),
            whole((B, CTX)),
            whole((5, MW)), whole((5, CTX)),
            whole((1, MW)), whole((1, CTX)),
            whole((MW, HID)), whole((CTX, HID)),
        ],
        out_specs=pl.BlockSpec((EB, HID), lambda i: (i, 0)),
        out_shape=jax.ShapeDtypeStruct((ESL, HID), jnp.float32),
    )(gh, pes, ea_p, ctx, *weights)


_BN_S = 1.0 / math.sqrt(1.0 + 1e-5)
_INV_SQRT2 = 1.0 / math.sqrt(2.0)


def _combine_body(h_ref, peb_ref, a0_ref, a1_ref, ctx_ref,
                  wnh_ref, wnpl_ref, wnph_ref, wnc_ref,
                  bn_ref, g_ref, bt_ref, hn_ref):
    h = h_ref[...]
    pes = peb_ref[...]
    pe_lo, pe_hi = _unpack_pe(pes[:, :PEP])
    gbf = pes[:, PEP:PEP + 1]
    oh = (gbf.astype(jnp.int32) == lax.broadcasted_iota(jnp.int32, (NB, B), 1)
          ).astype(jnp.float32)
    cproj = jnp.dot(ctx_ref[...], wnc_ref[...], preferred_element_type=jnp.float32)
    a0 = a0_ref[...]
    a1 = a1_ref[...]
    out = (jnp.dot(h, wnh_ref[...], preferred_element_type=jnp.float32)
           + jnp.dot(pe_lo, wnpl_ref[...], preferred_element_type=jnp.float32)
           + jnp.dot(pe_hi, wnph_ref[...], preferred_element_type=jnp.float32)
           + jnp.dot(oh, cproj, preferred_element_type=jnp.float32)
           + a0[0] + a0[1] + a1[0] + a1[1] + bn_ref[...])
    ob = out * _BN_S * g_ref[...] + bt_ref[...]
    gelu = 0.5 * ob * (1.0 + lax.erf(ob * _INV_SQRT2))
    hn_ref[...] = h + gelu


def _tc_combine(h, peb, aggr0, aggr1, ctx, wnh, wnpl, wnph, wnc, bn2, g2, bt2):
    grid = N // NB

    def whole(shape):
        return pl.BlockSpec(shape, lambda i: tuple(0 for _ in shape))

    return pl.pallas_call(
        _combine_body,
        grid=(grid,),
        in_specs=[
            pl.BlockSpec((NB, HID), lambda i: (i, 0)),
            pl.BlockSpec((NB, PEB_W), lambda i: (i, 0)),
            pl.BlockSpec((NC, NB, HID), lambda i: (0, i, 0)),
            pl.BlockSpec((NC, NB, HID), lambda i: (0, i, 0)),
            whole((B, CTX)),
            whole((HID, HID)), whole((PEP, HID)), whole((PEP, HID)),
            whole((CTX, HID)),
            whole((1, HID)), whole((1, HID)), whole((1, HID)),
        ],
        out_specs=pl.BlockSpec((NB, HID), lambda i: (i, 0)),
        out_shape=jax.ShapeDtypeStruct((N_PAD, HID), jnp.float32),
    )(h, peb, aggr0, aggr1, ctx, wnh, wnpl, wnph, wnc, bn2, g2, bt2)


def _final_body(h_ref, peb_ref, wl_ref, bl_ref, out_ref, acc_ref, cnt_ref):
    i = pl.program_id(0)

    @pl.when(i == 0)
    def _():
        acc_ref[...] = jnp.zeros_like(acc_ref)
        cnt_ref[...] = jnp.zeros_like(cnt_ref)

    h = h_ref[...]
    gbf = peb_ref[...][:, PEP:PEP + 1]
    oh = (gbf.astype(jnp.int32) == lax.broadcasted_iota(jnp.int32, (NB, B), 1)
          ).astype(jnp.float32)
    dn = (((0,), (0,)), ((), ()))
    acc_ref[...] += lax.dot_general(oh, h, dn, preferred_element_type=jnp.float32)
    cnt_ref[...] += lax.dot_general(oh, jnp.ones((NB, HID), jnp.float32), dn,
                                    preferred_element_type=jnp.float32)

    @pl.when(i == pl.num_programs(0) - 1)
    def _():
        out_ref[...] = (jnp.dot(acc_ref[...], wl_ref[...],
                                preferred_element_type=jnp.float32)
                        + cnt_ref[...][:, 0:1] * bl_ref[...])


def _tc_final(h, peb, wl, bl2):
    grid = N // NB
    return pl.pallas_call(
        _final_body,
        grid=(grid,),
        in_specs=[
            pl.BlockSpec((NB, HID), lambda i: (i, 0)),
            pl.BlockSpec((NB, PEB_W), lambda i: (i, 0)),
            pl.BlockSpec((HID, OUT), lambda i: (0, 0)),
            pl.BlockSpec((1, OUT), lambda i: (0, 0)),
        ],
        out_specs=pl.BlockSpec((B, OUT), lambda i: (0, 0)),
        out_shape=jax.ShapeDtypeStruct((B, OUT), jnp.float32),
        scratch_shapes=[
            pltpu.VMEM((B, HID), jnp.float32),
            pltpu.VMEM((B, HID), jnp.float32),
        ],
    )(h, peb, wl, bl2)


def kernel(x, pos, edge_index, edge_attr, batch, context_vector,
           W0, b0,
           Wn0, bn0, We0, be0, g0, bt0,
           Wn1, bn1, We1, be1, g1, bt1,
           Wn2, bn2, We2, be2, g2, bt2,
           Wl, bl):
    f32 = jnp.float32
    src = jnp.pad(edge_index[0].astype(jnp.int32), (0, E_PAD - E))
    dst = jnp.pad(edge_index[1].astype(jnp.int32), (0, E_PAD - E),
                  constant_values=DUMMY)
    ea_p = jnp.pad(edge_attr.astype(f32), ((0, E_PAD - E), (0, 0)))
    pos_p = jnp.pad(pos.astype(f32), ((0, 0), (0, 5)))
    x2 = x.reshape(N, 1).astype(jnp.int32)
    batch2 = batch.reshape(N, 1).astype(jnp.int32)
    zeros_rows = jnp.zeros((N_PAD, HID), f32)

    gather_full = _make_gather(E_PAD)
    gather_slab = _make_gather(ESL)
    scatter0 = _make_scatter(0)
    scatter1 = _make_scatter(ESL)

    h, peb = _tc_prep(x2, pos_p, batch2, W0.astype(f32), b0.reshape(1, HID))
    pes = gather_full(peb, src)
    src0 = lax.slice(src, (0,), (ESL,))
    src1 = lax.slice(src, (ESL,), (E_PAD,))

    layers = [(Wn0, bn0, We0, be0, g0, bt0),
              (Wn1, bn1, We1, be1, g1, bt1),
              (Wn2, bn2, We2, be2, g2, bt2)]
    for (Wn, bn, We, be, g, bt) in layers:
        z5_8 = jnp.zeros((5, 8), jnp.float32)
        z1_8 = jnp.zeros((1, 8), jnp.float32)
        z8_h = jnp.zeros((8, HID), jnp.float32)
        wem = jnp.concatenate(
            [We[:, :HID], We[:, HID:HID + PEP], z5_8,
             We[:, HID + PEP:HID + PED], z5_8], axis=1)
        bem = jnp.concatenate(
            [be[:HID], be[HID:HID + PEP], z1_8[0], be[HID + PEP:HID + PED],
             z1_8[0]]).reshape(1, MW)
        wnm = jnp.concatenate(
            [Wn[:HID], Wn[HID:HID + PEP], z8_h, Wn[HID + PEP:HID + PED], z8_h],
            axis=0)
        weights = (
            wem, We[:, HID + PED:],
            bem, be[HID + PED:].reshape(1, CTX),
            wnm, Wn[HID + PED:],
        )
        gh0 = gather_slab(h, src0)
        gh1 = gather_slab(h, src1)
        m0 = _tc_msg(0, gh0, pes, ea_p, context_vector, weights)
        m1 = _tc_msg(1, gh1, pes, ea_p, context_vector, weights)
        aggr0 = scatter0(m0, dst, zeros_rows)
        aggr1 = scatter1(m1, dst, zeros_rows)
        h = _tc_combine(h, peb, aggr0, aggr1, context_vector,
                        Wn[:HID], Wn[HID:HID + PEP], Wn[HID + PEP:HID + PED],
                        Wn[HID + PED:],
                        bn.reshape(1, HID), g.reshape(1, HID), bt.reshape(1, HID))

    return _tc_final(h, peb, Wl, bl.reshape(1, OUT))


# speedup vs baseline: 1.0956x; 1.0956x over previous
import math

import numpy as np
import jax
import jax.numpy as jnp
from jax import lax
from jax.experimental import pallas as pl
from jax.experimental.pallas import tpu as pltpu
from jax.experimental.pallas import tpu_sc as plsc

N = 10000
E = 160000
B = 64
CTX = 512
PED = 240
HID = 128
OUT = 1024

NC = 2
NS = 16
NW = NC * NS
CH = 128
E_PAD = 163840
ESL = E_PAD // 2
N_PAD = 10112
RPS = N_PAD // NS
DUMMY = N + 8

PEP = PED // 2
PEB_W = 128

NB = 2000
EB = 4096


def _sc_mesh():
    return plsc.VectorSubcoreMesh(core_axis_name="c", subcore_axis_name="s",
                                  num_cores=NC, num_subcores=NS)


def _make_gather(n_edges):
    epw = n_edges // NW
    nchunk = epw // CH
    assert nchunk % 2 == 0

    def body(table_hbm, idx_hbm, out_hbm, tab_sh, i0, i1, r0, r1, *sems):
        idxb = (i0, i1)
        rows = (r0, r1)
        isem = sems[0:2]
        gsem = sems[2:4]
        wsem = sems[4:6]
        c = lax.axis_index("c")
        s = lax.axis_index("s")
        base = (s * NC + c) * epw

        pltpu.sync_copy(table_hbm.at[pl.ds(s * RPS, RPS)],
                        tab_sh.at[pl.ds(s * RPS, RPS)])

        def issue_idx(ci, b):
            pltpu.async_copy(idx_hbm.at[pl.ds(base + ci * CH, CH)], idxb[b],
                             isem[b])

        def wait_idx(ci, b):
            pltpu.make_async_copy(idx_hbm.at[pl.ds(base + ci * CH, CH)], idxb[b],
                                  isem[b]).wait()

        def issue_gather(ci, b):
            pltpu.async_copy(tab_sh.at[idxb[b]], rows[b], gsem[b])

        def wait_gather(ci, b):
            pltpu.make_async_copy(tab_sh.at[idxb[b]], rows[b], gsem[b]).wait()

        def issue_wb(ci, b):
            pltpu.async_copy(rows[b], out_hbm.at[pl.ds(base + ci * CH, CH)],
                             wsem[b])

        def wait_wb(ci, b):
            pltpu.make_async_copy(rows[b], out_hbm.at[pl.ds(base + ci * CH, CH)],
                                  wsem[b]).wait()

        issue_idx(0, 0)
        issue_idx(1, 1)
        plsc.subcore_barrier()

        def loop(g):
            for b in range(2):
                ci = g + b

                @pl.when(ci >= 2)
                def _():
                    wait_wb(ci - 2, b)

                wait_idx(ci, b)
                issue_gather(ci, b)
                wait_gather(ci, b)
                issue_wb(ci, b)

                ci2 = ci + 2

                @pl.when(ci2 < nchunk)
                def _():
                    issue_idx(ci2, b)

        pl.loop(0, nchunk, step=2)(loop)
        wait_wb(nchunk - 2, 0)
        wait_wb(nchunk - 1, 1)

    kern = pl.kernel(
        body,
        out_type=jax.ShapeDtypeStruct((n_edges, PEB_W), jnp.float32),
        mesh=_sc_mesh(),
        scratch_types=(
            [pltpu.VMEM_SHARED((N_PAD, PEB_W), jnp.float32)]
            + [pltpu.VMEM((CH,), jnp.int32)] * 2
            + [pltpu.VMEM((CH, PEB_W), jnp.float32)] * 2
            + [pltpu.SemaphoreType.DMA] * 6
        ),
    )
    return kern


def _make_scatter(slab_off):
    epw = ESL // NW
    nchunk = epw // CH

    def body(m_hbm, dst_hbm, zeros_hbm, out_hbm, *refs):
        idxb = refs[0:3]
        rows = refs[3:6]
        acc_sh = refs[6]
        sems = refs[7:]
        isem = sems[0:3]
        lsem = sems[3:6]
        ssem = sems[6:9]
        c = lax.axis_index("c")
        s = lax.axis_index("s")
        pltpu.sync_copy(zeros_hbm.at[pl.ds(s * RPS, RPS)],
                        acc_sh.at[pl.ds(s * RPS, RPS)])
        base = c * (ESL // NC) + s * epw

        def issue_idx(ci, b):
            pltpu.async_copy(dst_hbm.at[pl.ds(slab_off + base + ci * CH, CH)],
                             idxb[b], isem[b])

        def wait_idx(ci, b):
            pltpu.make_async_copy(dst_hbm.at[pl.ds(slab_off + base + ci * CH, CH)],
                                  idxb[b], isem[b]).wait()

        def issue_load(ci, b):
            pltpu.async_copy(m_hbm.at[pl.ds(base + ci * CH, CH)], rows[b], lsem[b])

        def wait_load(ci, b):
            pltpu.make_async_copy(m_hbm.at[pl.ds(base + ci * CH, CH)], rows[b],
                                  lsem[b]).wait()

        def issue_scat(ci, b):
            pltpu.async_copy(rows[b], acc_sh.at[idxb[b]], ssem[b], add=True)

        def wait_scat(ci, b):
            pltpu.make_async_copy(rows[b], acc_sh.at[idxb[b]], ssem[b]).wait()

        issue_idx(0, 0)
        issue_load(0, 0)
        issue_idx(1, 1)
        issue_load(1, 1)
        plsc.subcore_barrier()

        def loop(g):
            for b in range(3):
                ci = g + b
                wait_idx(ci, b)
                wait_load(ci, b)
                issue_scat(ci, b)
                cl = ci + 2
                b2 = (b + 2) % 3

                @pl.when(cl < nchunk)
                def _():
                    @pl.when(cl >= 3)
                    def _():
                        wait_scat(cl - 3, b2)

                    issue_idx(cl, b2)
                    issue_load(cl, b2)

        pl.loop(0, nchunk - 2, step=3)(loop)
        wait_idx(nchunk - 2, 0)
        wait_load(nchunk - 2, 0)
        issue_scat(nchunk - 2, 0)
        wait_idx(nchunk - 1, 1)
        wait_load(nchunk - 1, 1)
        issue_scat(nchunk - 1, 1)
        wait_scat(nchunk - 3, 2)
        wait_scat(nchunk - 2, 0)
        wait_scat(nchunk - 1, 1)
        plsc.subcore_barrier()
        pltpu.sync_copy(acc_sh.at[pl.ds(s * RPS, RPS)],
                        out_hbm.at[c, pl.ds(s * RPS, RPS)])

    kern = pl.kernel(
        body,
        out_type=jax.ShapeDtypeStruct((NC, N_PAD, HID), jnp.float32),
        mesh=_sc_mesh(),
        scratch_types=(
            [pltpu.VMEM((CH,), jnp.int32)] * 3
            + [pltpu.VMEM((CH, HID), jnp.float32)] * 3
            + [pltpu.VMEM_SHARED((N_PAD, HID), jnp.float32)]
            + [pltpu.SemaphoreType.DMA] * 9
        ),
    )
    return kern


_HI_MASK = np.uint32(0xFFFF0000)
_RND = np.uint32(0x8000)


def _unpack_pe(words_f32):
    w = lax.bitcast_convert_type(words_f32, jnp.uint32)
    lo = lax.bitcast_convert_type(w & _HI_MASK, jnp.float32)
    hi = lax.bitcast_convert_type(lax.shift_left(w, np.uint32(16)), jnp.float32)
    return lo, hi


def _prep_body(x_ref, pos_ref, batch_ref, w0_ref, b0_ref, h0_ref, peb_ref):
    xi = x_ref[...]
    oh = (xi == lax.broadcasted_iota(jnp.int32, (NB, 118), 1)).astype(jnp.float32)
    h0_ref[...] = jnp.dot(oh, w0_ref[...], preferred_element_type=jnp.float32) + b0_ref[...]

    dt = jnp.exp(lax.broadcasted_iota(jnp.int32, (1, PED // 6), 1).astype(jnp.float32)
                 * (-math.log(10000.0) / (PED // 2)))
    pos = pos_ref[...]
    parts = []
    for i in range(3):
        sarg = pos[:, i:i + 1] * dt
        parts.append(jnp.sin(sarg))
        parts.append(jnp.cos(sarg))
    lo = jnp.concatenate(parts[:3], axis=1)
    hi = jnp.concatenate(parts[3:], axis=1)
    ulo = lax.bitcast_convert_type(lo, jnp.uint32)
    uhi = lax.bitcast_convert_type(hi, jnp.uint32)
    word = ((ulo + _RND) & _HI_MASK) | lax.shift_right_logical(uhi + _RND,
                                                               np.uint32(16))
    packed = lax.bitcast_convert_type(word, jnp.float32)
    bf = batch_ref[...].astype(jnp.float32)
    pad = jnp.zeros((NB, PEB_W - PEP - 1), jnp.float32)
    peb_ref[...] = jnp.concatenate([packed, bf, pad], axis=1)


def _tc_prep(x, pos_p, batch2, w0, b02):
    grid = N // NB
    return pl.pallas_call(
        _prep_body,
        grid=(grid,),
        in_specs=[
            pl.BlockSpec((NB, 1), lambda i: (i, 0)),
            pl.BlockSpec((NB, 8), lambda i: (i, 0)),
            pl.BlockSpec((NB, 1), lambda i: (i, 0)),
            pl.BlockSpec((118, HID), lambda i: (0, 0)),
            pl.BlockSpec((1, HID), lambda i: (0, 0)),
        ],
        out_specs=[
            pl.BlockSpec((NB, HID), lambda i: (i, 0)),
            pl.BlockSpec((NB, PEB_W), lambda i: (i, 0)),
        ],
        out_shape=[
            jax.ShapeDtypeStruct((N_PAD, HID), jnp.float32),
            jax.ShapeDtypeStruct((N_PAD, PEB_W), jnp.float32),
        ],
    )(x, pos_p, batch2, w0, b02)


def _bdot(a, b):
    return jnp.dot(a.astype(jnp.bfloat16), b.astype(jnp.bfloat16),
                   preferred_element_type=jnp.float32)


def _msg_body(gh_ref, pes_ref, ea_ref, ctx_ref,
              weh_ref, wepl_ref, weph_ref, wec_ref,
              beh_ref, bepl_ref, beph_ref, bec_ref,
              wnh_ref, wnpl_ref, wnph_ref, wnc_ref, m_ref):
    ea = ea_ref[...]
    gh = gh_ref[...]
    pes = pes_ref[...]
    pe_lo, pe_hi = _unpack_pe(pes[:, :PEP])
    gbf = pes[:, PEP:PEP + 1]

    pre_h = jax.nn.relu(gh + jnp.dot(ea, weh_ref[...], preferred_element_type=jnp.float32)
                        + beh_ref[...])
    pre_pl = jax.nn.relu(pe_lo + jnp.dot(ea, wepl_ref[...], preferred_element_type=jnp.float32)
                         + bepl_ref[...])
    pre_ph = jax.nn.relu(pe_hi + jnp.dot(ea, weph_ref[...], preferred_element_type=jnp.float32)
                         + beph_ref[...])
    oh = (gbf.astype(jnp.int32) == lax.broadcasted_iota(jnp.int32, (EB, B), 1)
          ).astype(jnp.float32)
    cg = _bdot(oh, ctx_ref[...])
    pre_c = jax.nn.relu(cg + jnp.dot(ea, wec_ref[...], preferred_element_type=jnp.float32)
                        + bec_ref[...])

    m_ref[...] = (_bdot(pre_h, wnh_ref[...]) + _bdot(pre_pl, wnpl_ref[...])
                  + _bdot(pre_ph, wnph_ref[...]) + _bdot(pre_c, wnc_ref[...]))


def _tc_msg(slab, gh, pes, ea_p, ctx, weights):
    grid = ESL // EB
    blk0 = slab * (ESL // EB)

    def whole(shape):
        return pl.BlockSpec(shape, lambda i: tuple(0 for _ in shape))

    return pl.pallas_call(
        _msg_body,
        grid=(grid,),
        in_specs=[
            pl.BlockSpec((EB, HID), lambda i: (i, 0)),
            pl.BlockSpec((EB, PEB_W), lambda i: (i + blk0, 0)),
            pl.BlockSpec((EB, 5), lambda i: (i + blk0, 0)),
            whole((B, CTX)),
            whole((5, HID)), whole((5, PEP)), whole((5, PEP)), whole((5, CTX)),
            whole((1, HID)), whole((1, PEP)), whole((1, PEP)), whole((1, CTX)),
            whole((HID, HID)), whole((PEP, HID)), whole((PEP, HID)),
            whole((CTX, HID)),
        ],
        out_specs=pl.BlockSpec((EB, HID), lambda i: (i, 0)),
        out_shape=jax.ShapeDtypeStruct((ESL, HID), jnp.float32),
    )(gh, pes, ea_p, ctx, *weights)


_BN_S = 1.0 / math.sqrt(1.0 + 1e-5)
_INV_SQRT2 = 1.0 / math.sqrt(2.0)


def _combine_pre_body(h_ref, peb_ref, ctx_ref,
                      wnh_ref, wnpl_ref, wnph_ref, wnc_ref, bn_ref, cp_ref):
    h = h_ref[...]
    pes = peb_ref[...]
    pe_lo, pe_hi = _unpack_pe(pes[:, :PEP])
    gbf = pes[:, PEP:PEP + 1]
    oh = (gbf.astype(jnp.int32) == lax.broadcasted_iota(jnp.int32, (NB, B), 1)
          ).astype(jnp.float32)
    cproj = jnp.dot(ctx_ref[...], wnc_ref[...], preferred_element_type=jnp.float32)
    cp_ref[...] = (jnp.dot(h, wnh_ref[...], preferred_element_type=jnp.float32)
                   + jnp.dot(pe_lo, wnpl_ref[...], preferred_element_type=jnp.float32)
                   + jnp.dot(pe_hi, wnph_ref[...], preferred_element_type=jnp.float32)
                   + jnp.dot(oh, cproj, preferred_element_type=jnp.float32)
                   + bn_ref[...])


def _tc_combine_pre(h, peb, ctx, wnh, wnpl, wnph, wnc, bn2):
    grid = N // NB

    def whole(shape):
        return pl.BlockSpec(shape, lambda i: tuple(0 for _ in shape))

    return pl.pallas_call(
        _combine_pre_body,
        grid=(grid,),
        in_specs=[
            pl.BlockSpec((NB, HID), lambda i: (i, 0)),
            pl.BlockSpec((NB, PEB_W), lambda i: (i, 0)),
            whole((B, CTX)),
            whole((HID, HID)), whole((PEP, HID)), whole((PEP, HID)),
            whole((CTX, HID)),
            whole((1, HID)),
        ],
        out_specs=pl.BlockSpec((NB, HID), lambda i: (i, 0)),
        out_shape=jax.ShapeDtypeStruct((N_PAD, HID), jnp.float32),
    )(h, peb, ctx, wnh, wnpl, wnph, wnc, bn2)


def _combine_fin_body(h_ref, cp_ref, a0_ref, a1_ref, g_ref, bt_ref, hn_ref):
    a0 = a0_ref[...]
    a1 = a1_ref[...]
    out = cp_ref[...] + a0[0] + a0[1] + a1[0] + a1[1]
    ob = out * _BN_S * g_ref[...] + bt_ref[...]
    gelu = 0.5 * ob * (1.0 + lax.erf(ob * _INV_SQRT2))
    hn_ref[...] = h_ref[...] + gelu


def _tc_combine_fin(h, cpre, aggr0, aggr1, g2, bt2):
    grid = N // NB

    def whole(shape):
        return pl.BlockSpec(shape, lambda i: tuple(0 for _ in shape))

    return pl.pallas_call(
        _combine_fin_body,
        grid=(grid,),
        in_specs=[
            pl.BlockSpec((NB, HID), lambda i: (i, 0)),
            pl.BlockSpec((NB, HID), lambda i: (i, 0)),
            pl.BlockSpec((NC, NB, HID), lambda i: (0, i, 0)),
            pl.BlockSpec((NC, NB, HID), lambda i: (0, i, 0)),
            whole((1, HID)), whole((1, HID)),
        ],
        out_specs=pl.BlockSpec((NB, HID), lambda i: (i, 0)),
        out_shape=jax.ShapeDtypeStruct((N_PAD, HID), jnp.float32),
    )(h, cpre, aggr0, aggr1, g2, bt2)


def _final_body(h_ref, peb_ref, wl_ref, bl_ref, out_ref, acc_ref, cnt_ref):
    i = pl.program_id(0)

    @pl.when(i == 0)
    def _():
        acc_ref[...] = jnp.zeros_like(acc_ref)
        cnt_ref[...] = jnp.zeros_like(cnt_ref)

    h = h_ref[...]
    gbf = peb_ref[...][:, PEP:PEP + 1]
    oh = (gbf.astype(jnp.int32) == lax.broadcasted_iota(jnp.int32, (NB, B), 1)
          ).astype(jnp.float32)
    dn = (((0,), (0,)), ((), ()))
    acc_ref[...] += lax.dot_general(oh, h, dn, preferred_element_type=jnp.float32)
    cnt_ref[...] += lax.dot_general(oh, jnp.ones((NB, HID), jnp.float32), dn,
                                    preferred_element_type=jnp.float32)

    @pl.when(i == pl.num_programs(0) - 1)
    def _():
        out_ref[...] = (jnp.dot(acc_ref[...], wl_ref[...],
                                preferred_element_type=jnp.float32)
                        + cnt_ref[...][:, 0:1] * bl_ref[...])


def _tc_final(h, peb, wl, bl2):
    grid = N // NB
    return pl.pallas_call(
        _final_body,
        grid=(grid,),
        in_specs=[
            pl.BlockSpec((NB, HID), lambda i: (i, 0)),
            pl.BlockSpec((NB, PEB_W), lambda i: (i, 0)),
            pl.BlockSpec((HID, OUT), lambda i: (0, 0)),
            pl.BlockSpec((1, OUT), lambda i: (0, 0)),
        ],
        out_specs=pl.BlockSpec((B, OUT), lambda i: (0, 0)),
        out_shape=jax.ShapeDtypeStruct((B, OUT), jnp.float32),
        scratch_shapes=[
            pltpu.VMEM((B, HID), jnp.float32),
            pltpu.VMEM((B, HID), jnp.float32),
        ],
    )(h, peb, wl, bl2)


def kernel(x, pos, edge_index, edge_attr, batch, context_vector,
           W0, b0,
           Wn0, bn0, We0, be0, g0, bt0,
           Wn1, bn1, We1, be1, g1, bt1,
           Wn2, bn2, We2, be2, g2, bt2,
           Wl, bl):
    f32 = jnp.float32
    src = jnp.pad(edge_index[0].astype(jnp.int32), (0, E_PAD - E))
    dst = jnp.pad(edge_index[1].astype(jnp.int32), (0, E_PAD - E),
                  constant_values=DUMMY)
    ea_p = jnp.pad(edge_attr.astype(f32), ((0, E_PAD - E), (0, 0)))
    pos_p = jnp.pad(pos.astype(f32), ((0, 0), (0, 5)))
    x2 = x.reshape(N, 1).astype(jnp.int32)
    batch2 = batch.reshape(N, 1).astype(jnp.int32)
    zeros_rows = jnp.zeros((N_PAD, HID), f32)

    gather_full = _make_gather(E_PAD)
    gather_slab = _make_gather(ESL)
    scatter0 = _make_scatter(0)
    scatter1 = _make_scatter(ESL)

    h, peb = _tc_prep(x2, pos_p, batch2, W0.astype(f32), b0.reshape(1, HID))
    pes = gather_full(peb, src)
    src0 = lax.slice(src, (0,), (ESL,))
    src1 = lax.slice(src, (ESL,), (E_PAD,))

    layers = [(Wn0, bn0, We0, be0, g0, bt0),
              (Wn1, bn1, We1, be1, g1, bt1),
              (Wn2, bn2, We2, be2, g2, bt2)]
    for (Wn, bn, We, be, g, bt) in layers:
        weights = (
            We[:, :HID], We[:, HID:HID + PEP], We[:, HID + PEP:HID + PED],
            We[:, HID + PED:],
            be[:HID].reshape(1, HID),
            be[HID:HID + PEP].reshape(1, PEP),
            be[HID + PEP:HID + PED].reshape(1, PEP),
            be[HID + PED:].reshape(1, CTX),
            Wn[:HID], Wn[HID:HID + PEP], Wn[HID + PEP:HID + PED],
            Wn[HID + PED:],
        )
        gh0 = gather_slab(h, src0)
        gh1 = gather_slab(h, src1)
        m0 = _tc_msg(0, gh0, pes, ea_p, context_vector, weights)
        m1 = _tc_msg(1, gh1, pes, ea_p, context_vector, weights)
        aggr0 = scatter0(m0, dst, zeros_rows)
        cpre = _tc_combine_pre(h, peb, context_vector,
                               Wn[:HID], Wn[HID:HID + PEP],
                               Wn[HID + PEP:HID + PED], Wn[HID + PED:],
                               bn.reshape(1, HID))
        aggr1 = scatter1(m1, dst, zeros_rows)
        h = _tc_combine_fin(h, cpre, aggr0, aggr1,
                            g.reshape(1, HID), bt.reshape(1, HID))

    return _tc_final(h, peb, Wl, bl.reshape(1, OUT))
